# baseline (speedup 1.0000x reference)
import functools

import jax
import jax.numpy as jnp
from jax import lax
from jax.experimental import pallas as pl
from jax.experimental.pallas import tpu as pltpu
from jax.experimental.pallas import tpu_sc as plsc

EMBED_DIM = 1024
NBINS = 6
BATCH = 16384

_info = plsc.get_sparse_core_info()
_NC, _NS = _info.num_cores, _info.num_subcores
_NW = _NC * _NS
_B_PER_W = BATCH // _NW
_CHUNK = 32
_N_CHUNKS = _B_PER_W // _CHUNK


def _sc_embed(idx_h, idx_s, idx_t, helix_table, sheet_table, turns_table):
    mesh = plsc.VectorSubcoreMesh(core_axis_name="c", subcore_axis_name="s")
    row = jax.ShapeDtypeStruct((BATCH, EMBED_DIM), jnp.float32)

    @functools.partial(
        pl.kernel,
        out_type=(row, row, row),
        mesh=mesh,
        scratch_types=[
            pltpu.VMEM((_B_PER_W,), jnp.int32),
            pltpu.VMEM((_CHUNK, EMBED_DIM), jnp.float32),
            pltpu.SemaphoreType.DMA,
        ],
    )
    def body(i0, i1, i2, t0, t1, t2, o0, o1, o2, idx_v, rows_v, sem):
        wid = lax.axis_index("s") * _NC + lax.axis_index("c")
        base = wid * _B_PER_W
        for idx_hbm, tab_hbm, out_hbm in ((i0, t0, o0), (i1, t1, o1), (i2, t2, o2)):
            pltpu.sync_copy(idx_hbm.at[pl.ds(base, _B_PER_W)], idx_v)

            @pl.loop(0, _N_CHUNKS)
            def _chunk(c):
                off = c * _CHUNK
                pltpu.async_copy(
                    tab_hbm.at[idx_v.at[pl.ds(off, _CHUNK)]], rows_v, sem
                ).wait()
                pltpu.sync_copy(rows_v, out_hbm.at[pl.ds(base + off, _CHUNK)])

    return body(idx_h, idx_s, idx_t, helix_table, sheet_table, turns_table)


def kernel(x, helix_table, sheet_table, turns_table):
    xi = x.astype(jnp.int32)
    return _sc_embed(
        xi[:, 0], xi[:, 1], xi[:, 2], helix_table, sheet_table, turns_table
    )


# speedup vs baseline: 1.0132x; 1.0132x over previous
import functools

import jax
import jax.numpy as jnp
from jax import lax
from jax.experimental import pallas as pl
from jax.experimental.pallas import tpu as pltpu
from jax.experimental.pallas import tpu_sc as plsc

EMBED_DIM = 1024
NBINS = 6
BATCH = 16384

_info = plsc.get_sparse_core_info()
_NC, _NS = _info.num_cores, _info.num_subcores
_NW = _NC * _NS
_B_PER_W = BATCH // _NW
_CHUNK = 16
_NBUF = 4
_N_CHUNKS = _B_PER_W // _CHUNK


def _sc_embed(idx_h, idx_s, idx_t, helix_table, sheet_table, turns_table):
    mesh = plsc.VectorSubcoreMesh(core_axis_name="c", subcore_axis_name="s")
    row = jax.ShapeDtypeStruct((BATCH, EMBED_DIM), jnp.float32)

    @functools.partial(
        pl.kernel,
        out_type=(row, row, row),
        mesh=mesh,
        scratch_types=[
            pltpu.VMEM((_B_PER_W,), jnp.int32),
            [pltpu.VMEM((_CHUNK, EMBED_DIM), jnp.float32) for _ in range(_NBUF)],
            [pltpu.SemaphoreType.DMA for _ in range(_NBUF)],
            [pltpu.SemaphoreType.DMA for _ in range(_NBUF)],
        ],
    )
    def body(i0, i1, i2, t0, t1, t2, o0, o1, o2, idx_v, rows, gsem, ssem):
        wid = lax.axis_index("s") * _NC + lax.axis_index("c")
        base = wid * _B_PER_W

        for idx_hbm, tab_hbm, out_hbm in ((i0, t0, o0), (i1, t1, o1), (i2, t2, o2)):
            pltpu.sync_copy(idx_hbm.at[pl.ds(base, _B_PER_W)], idx_v)

            def gather(n, b):
                pltpu.async_copy(
                    tab_hbm.at[idx_v.at[pl.ds(n * _CHUNK, _CHUNK)]], rows[b], gsem[b]
                )

            def gather_wait(b):
                pltpu.make_async_copy(
                    out_hbm.at[pl.ds(0, _CHUNK)], rows[b], gsem[b]
                ).wait()

            def scatter(n, b):
                pltpu.async_copy(
                    rows[b], out_hbm.at[pl.ds(base + n * _CHUNK, _CHUNK)], ssem[b]
                )

            def scatter_wait(b):
                pltpu.make_async_copy(
                    rows[b], out_hbm.at[pl.ds(0, _CHUNK)], ssem[b]
                ).wait()

            for b in range(_NBUF):
                gather(b, b)

            @pl.loop(0, _N_CHUNKS - _NBUF, step=_NBUF)
            def _steady(j):
                for b in range(_NBUF):
                    gather_wait(b)
                    scatter(j + b, b)
                for b in range(_NBUF):
                    scatter_wait(b)
                    gather(j + _NBUF + b, b)

            for b in range(_NBUF):
                gather_wait(b)
                scatter(_N_CHUNKS - _NBUF + b, b)
            for b in range(_NBUF):
                scatter_wait(b)

    return body(idx_h, idx_s, idx_t, helix_table, sheet_table, turns_table)


def kernel(x, helix_table, sheet_table, turns_table):
    xi = x.astype(jnp.int32)
    return _sc_embed(
        xi[:, 0], xi[:, 1], xi[:, 2], helix_table, sheet_table, turns_table
    )


# speedup vs baseline: 1.1476x; 1.1327x over previous
import functools

import jax
import jax.numpy as jnp
from jax import lax
from jax.experimental import pallas as pl
from jax.experimental.pallas import tpu as pltpu
from jax.experimental.pallas import tpu_sc as plsc

EMBED_DIM = 1024
NBINS = 6
BATCH = 16384

_info = plsc.get_sparse_core_info()
_NC, _NS = _info.num_cores, _info.num_subcores
_NW = _NC * _NS
_B_PER_W = BATCH // _NW
_CHUNK = 16
_NBUF = 2
_N_CHUNKS = _B_PER_W // _CHUNK
_CHUNK_ELEMS = _CHUNK * EMBED_DIM


def _sc_embed(idx_h, idx_s, idx_t, helix_flat, sheet_flat, turns_flat):
    mesh = plsc.VectorSubcoreMesh(core_axis_name="c", subcore_axis_name="s")
    flat = jax.ShapeDtypeStruct((BATCH * EMBED_DIM,), jnp.float32)

    @functools.partial(
        pl.kernel,
        out_type=(flat, flat, flat),
        mesh=mesh,
        compiler_params=pltpu.CompilerParams(needs_layout_passes=False),
        scratch_types=[
            pltpu.VMEM((_B_PER_W,), jnp.int32),
            [pltpu.VMEM((NBINS * EMBED_DIM,), jnp.float32) for _ in range(3)],
            [pltpu.VMEM((_CHUNK_ELEMS,), jnp.float32) for _ in range(_NBUF)],
            [pltpu.SemaphoreType.DMA for _ in range(_NBUF)],
        ],
    )
    def body(i0, i1, i2, t0, t1, t2, o0, o1, o2, idx_v, tab, stage, ssem):
        wid = lax.axis_index("s") * _NC + lax.axis_index("c")
        base = wid * _B_PER_W
        iota = lax.iota(jnp.int32, 16)

        for tab_hbm, tab_v in ((t0, tab[0]), (t1, tab[1]), (t2, tab[2])):
            pltpu.sync_copy(tab_hbm, tab_v)

        for idx_hbm, tab_v, out_hbm in (
            (i0, tab[0], o0), (i1, tab[1], o1), (i2, tab[2], o2)
        ):
            pltpu.sync_copy(idx_hbm.at[pl.ds(base, _B_PER_W)], idx_v)

            def compute(n, b):
                @pl.loop(0, _CHUNK)
                def _row(j):
                    rowv = plsc.load_gather(
                        idx_v, [jnp.full((16,), n * _CHUNK, jnp.int32) + j]
                    )
                    src0 = rowv * EMBED_DIM + iota
                    dst0 = j * EMBED_DIM + iota

                    @pl.loop(0, EMBED_DIM // 16, init_carry=(src0, dst0), unroll=8)
                    def _col(c, carry):
                        src, dst = carry
                        plsc.store_scatter(
                            stage[b], [dst], plsc.load_gather(tab_v, [src])
                        )
                        return (src + 16, dst + 16)

            def scatter(n, b):
                pltpu.async_copy(
                    stage[b],
                    out_hbm.at[pl.ds((base + n * _CHUNK) * EMBED_DIM, _CHUNK_ELEMS)],
                    ssem[b],
                )

            def scatter_wait(b):
                pltpu.make_async_copy(
                    stage[b], out_hbm.at[pl.ds(0, _CHUNK_ELEMS)], ssem[b]
                ).wait()

            @pl.loop(0, _N_CHUNKS, step=_NBUF)
            def _steady(j):
                for b in range(_NBUF):

                    @pl.when(j > 0)
                    def _drain():
                        scatter_wait(b)

                    compute(j + b, b)
                    scatter(j + b, b)

            for b in range(_NBUF):
                scatter_wait(b)

    return body(idx_h, idx_s, idx_t, helix_flat, sheet_flat, turns_flat)


def kernel(x, helix_table, sheet_table, turns_table):
    xi = x.astype(jnp.int32)
    o0, o1, o2 = _sc_embed(
        xi[:, 0],
        xi[:, 1],
        xi[:, 2],
        helix_table.reshape(-1),
        sheet_table.reshape(-1),
        turns_table.reshape(-1),
    )
    return (
        o0.reshape(BATCH, EMBED_DIM),
        o1.reshape(BATCH, EMBED_DIM),
        o2.reshape(BATCH, EMBED_DIM),
    )
